# baseline (speedup 1.0000x reference)
import functools

import jax
import jax.numpy as jnp
from jax import lax
from jax.experimental import pallas as pl
from jax.experimental.pallas import tpu as pltpu
from jax.experimental.pallas import tpu_sc as plsc

NT = 65536
D = 128
NC = 2
NS = 16
NW = NC * NS
TPW = NT // NW
NBUF = 4
CHUNK = 64
NCHUNK = TPW // CHUNK
GROUPS = CHUNK // 16


def _sc_z(x, a_idx, p_idx, n_idx):
    mesh = plsc.VectorSubcoreMesh(core_axis_name="c", subcore_axis_name="s")

    @functools.partial(
        pl.kernel,
        out_type=jax.ShapeDtypeStruct((NT,), jnp.float32),
        mesh=mesh,
        compiler_params=pltpu.CompilerParams(needs_layout_passes=False),
        scratch_types=[
            pltpu.VMEM((TPW,), jnp.int32),
            pltpu.VMEM((TPW,), jnp.int32),
            pltpu.VMEM((TPW,), jnp.int32),
            [pltpu.VMEM((CHUNK, D), jnp.float32) for _ in range(NBUF)],
            [pltpu.VMEM((CHUNK, D), jnp.float32) for _ in range(NBUF)],
            [pltpu.VMEM((CHUNK, D), jnp.float32) for _ in range(NBUF)],
            [pltpu.VMEM((CHUNK,), jnp.float32) for _ in range(NBUF)],
            pltpu.VMEM((GROUPS, 16, 17), jnp.float32),
            [pltpu.SemaphoreType.DMA for _ in range(NBUF)],
            [pltpu.SemaphoreType.DMA for _ in range(NBUF)],
        ],
    )
    def sc_kernel(x_hbm, a_hbm, p_hbm, n_hbm, z_hbm,
                  ia, ip, ineg, ra, rp, rn, zb, zvecs, sems, zsems):
        wid = lax.axis_index("s") * NC + lax.axis_index("c")
        base = wid * TPW

        pltpu.sync_copy(a_hbm.at[pl.ds(base, TPW)], ia)
        pltpu.sync_copy(p_hbm.at[pl.ds(base, TPW)], ip)
        pltpu.sync_copy(n_hbm.at[pl.ds(base, TPW)], ineg)

        def issue(ci, b):
            off = ci * CHUNK
            pltpu.async_copy(x_hbm.at[ia.at[pl.ds(off, CHUNK)]], ra[b], sems[b])
            pltpu.async_copy(x_hbm.at[ip.at[pl.ds(off, CHUNK)]], rp[b], sems[b])
            pltpu.async_copy(x_hbm.at[ineg.at[pl.ds(off, CHUNK)]], rn[b], sems[b])

        def drain(b):
            pltpu.make_async_copy(x_hbm.at[ia.at[pl.ds(0, CHUNK)]], ra[b], sems[b]).wait()
            pltpu.make_async_copy(x_hbm.at[ip.at[pl.ds(0, CHUNK)]], rp[b], sems[b]).wait()
            pltpu.make_async_copy(x_hbm.at[ineg.at[pl.ds(0, CHUNK)]], rn[b], sems[b]).wait()

        def compute(ci, b):
            @pl.when(ci >= NBUF)
            def _():
                pltpu.make_async_copy(
                    zb[b], z_hbm.at[pl.ds(base, CHUNK)], zsems[b]).wait()

            def group_body(g, c2):
                zero = jnp.zeros((16,), jnp.float32)
                acc_ap = [zero] * 16
                acc_an = [zero] * 16
                for k in range(D // 16):
                    sl = pl.ds(k * 16, 16)
                    for u in range(16):
                        t = g * 16 + u
                        va = ra[b][t, sl]
                        vp = rp[b][t, sl]
                        vn = rn[b][t, sl]
                        tp = va - vp
                        tn = va - vn
                        acc_ap[u] = acc_ap[u] + tp * tp
                        acc_an[u] = acc_an[u] + tn * tn
                for u in range(16):
                    zvecs[g, u, pl.ds(0, 16)] = acc_ap[u] - acc_an[u]
                rowi = lax.iota(jnp.int32, 16)
                gv = jnp.full((16,), 0, jnp.int32) + g
                vals = [
                    plsc.load_gather(
                        zvecs, [gv, rowi, jnp.full((16,), j, jnp.int32)])
                    for j in range(16)
                ]
                while len(vals) > 1:
                    vals = [vals[i] + vals[i + 1] for i in range(0, len(vals), 2)]
                zb[b][pl.ds(g * 16, 16)] = vals[0]
                return c2

            lax.fori_loop(0, GROUPS, group_body, 0)
            pltpu.async_copy(
                zb[b], z_hbm.at[pl.ds(base + ci * CHUNK, CHUNK)], zsems[b])

        for ci in range(NBUF - 1):
            issue(ci, ci)

        def ring_body(i, carry):
            for b in range(NBUF):
                ci = i * NBUF + b

                @pl.when(ci + NBUF - 1 < NCHUNK)
                def _():
                    issue(ci + NBUF - 1, (b + NBUF - 1) % NBUF)

                drain(b)
                compute(ci, b)
            return carry

        lax.fori_loop(0, NCHUNK // NBUF, ring_body, 0)
        for b in range(NBUF):
            pltpu.make_async_copy(
                zb[b], z_hbm.at[pl.ds(base, CHUNK)], zsems[b]).wait()

    return sc_kernel(x, a_idx, p_idx, n_idx)


def _tc_loss(z):
    def body(z_ref, o_ref):
        sp = jnp.log1p(jnp.exp(z_ref[...]))
        o_ref[0, 0] = jnp.sum(sp) * (1.0 / NT)

    out = pl.pallas_call(
        body,
        out_shape=jax.ShapeDtypeStruct((1, 1), jnp.float32),
        in_specs=[pl.BlockSpec(memory_space=pltpu.VMEM)],
        out_specs=pl.BlockSpec(memory_space=pltpu.SMEM),
    )(z.reshape(NT // 128, 128))
    return out.reshape(1)


def kernel(x, triplets):
    tri = triplets.astype(jnp.int32)
    z = _sc_z(x, tri[:, 0], tri[:, 1], tri[:, 2])
    return _tc_loss(z)


# speedup vs baseline: 1.1227x; 1.1227x over previous
import functools

import jax
import jax.numpy as jnp
from jax import lax
from jax.experimental import pallas as pl
from jax.experimental.pallas import tpu as pltpu
from jax.experimental.pallas import tpu_sc as plsc

NT = 65536
D = 128
NC = 2
NS = 16
NW = NC * NS
TPW = NT // NW
NBUF = 2
CHUNK = 128
NCHUNK = TPW // CHUNK
GROUPS = CHUNK // 16


def _sc_z(x, a_idx, p_idx, n_idx):
    mesh = plsc.VectorSubcoreMesh(core_axis_name="c", subcore_axis_name="s")

    @functools.partial(
        pl.kernel,
        out_type=jax.ShapeDtypeStruct((NT,), jnp.float32),
        mesh=mesh,
        compiler_params=pltpu.CompilerParams(needs_layout_passes=False),
        scratch_types=[
            pltpu.VMEM((TPW,), jnp.int32),
            pltpu.VMEM((TPW,), jnp.int32),
            pltpu.VMEM((TPW,), jnp.int32),
            [pltpu.VMEM((CHUNK, D), jnp.float32) for _ in range(NBUF)],
            [pltpu.VMEM((CHUNK, D), jnp.float32) for _ in range(NBUF)],
            [pltpu.VMEM((CHUNK, D), jnp.float32) for _ in range(NBUF)],
            [pltpu.VMEM((CHUNK,), jnp.float32) for _ in range(NBUF)],
            pltpu.VMEM((16, 17), jnp.float32),
            [pltpu.SemaphoreType.DMA for _ in range(NBUF)],
            [pltpu.SemaphoreType.DMA for _ in range(NBUF)],
        ],
    )
    def sc_kernel(x_hbm, a_hbm, p_hbm, n_hbm, z_hbm,
                  ia, ip, ineg, ra, rp, rn, zb, zvecs, sems, zsems):
        wid = lax.axis_index("s") * NC + lax.axis_index("c")
        base = wid * TPW

        pltpu.sync_copy(a_hbm.at[pl.ds(base, TPW)], ia)
        pltpu.sync_copy(p_hbm.at[pl.ds(base, TPW)], ip)
        pltpu.sync_copy(n_hbm.at[pl.ds(base, TPW)], ineg)

        def issue(ci, b):
            off = ci * CHUNK
            pltpu.async_copy(x_hbm.at[ia.at[pl.ds(off, CHUNK)]], ra[b], sems[b])
            pltpu.async_copy(x_hbm.at[ip.at[pl.ds(off, CHUNK)]], rp[b], sems[b])
            pltpu.async_copy(x_hbm.at[ineg.at[pl.ds(off, CHUNK)]], rn[b], sems[b])

        def drain(b):
            pltpu.make_async_copy(x_hbm.at[ia.at[pl.ds(0, CHUNK)]], ra[b], sems[b]).wait()
            pltpu.make_async_copy(x_hbm.at[ip.at[pl.ds(0, CHUNK)]], rp[b], sems[b]).wait()
            pltpu.make_async_copy(x_hbm.at[ineg.at[pl.ds(0, CHUNK)]], rn[b], sems[b]).wait()

        def compute(ci, b):
            @pl.when(ci >= NBUF)
            def _():
                pltpu.make_async_copy(
                    zb[b], z_hbm.at[pl.ds(base, CHUNK)], zsems[b]).wait()

            def group_body(g, c2):
                zero = jnp.zeros((16,), jnp.float32)
                acc_ap = [zero] * 16
                acc_an = [zero] * 16
                for k in range(D // 16):
                    sl = pl.ds(k * 16, 16)
                    for u in range(16):
                        t = g * 16 + u
                        va = ra[b][t, sl]
                        vp = rp[b][t, sl]
                        vn = rn[b][t, sl]
                        tp = va - vp
                        tn = va - vn
                        acc_ap[u] = acc_ap[u] + tp * tp
                        acc_an[u] = acc_an[u] + tn * tn
                for u in range(16):
                    zvecs[u, pl.ds(0, 16)] = acc_ap[u] - acc_an[u]
                rowi = lax.iota(jnp.int32, 16)
                vals = [
                    plsc.load_gather(zvecs, [rowi, jnp.full((16,), j, jnp.int32)])
                    for j in range(16)
                ]
                while len(vals) > 1:
                    vals = [vals[i] + vals[i + 1] for i in range(0, len(vals), 2)]
                zb[b][pl.ds(g * 16, 16)] = vals[0]
                return c2

            lax.fori_loop(0, GROUPS, group_body, 0)
            pltpu.async_copy(
                zb[b], z_hbm.at[pl.ds(base + ci * CHUNK, CHUNK)], zsems[b])

        for ci in range(NBUF - 1):
            issue(ci, ci)

        def ring_body(i, carry):
            for b in range(NBUF):
                ci = i * NBUF + b

                @pl.when(ci + NBUF - 1 < NCHUNK)
                def _():
                    issue(ci + NBUF - 1, (b + NBUF - 1) % NBUF)

                drain(b)
                compute(ci, b)
            return carry

        lax.fori_loop(0, NCHUNK // NBUF, ring_body, 0)
        for b in range(NBUF):
            pltpu.make_async_copy(
                zb[b], z_hbm.at[pl.ds(base, CHUNK)], zsems[b]).wait()

    return sc_kernel(x, a_idx, p_idx, n_idx)


def _tc_loss(z):
    def body(z_ref, o_ref):
        sp = jnp.log1p(jnp.exp(z_ref[...]))
        o_ref[0, 0] = jnp.sum(sp) * (1.0 / NT)

    out = pl.pallas_call(
        body,
        out_shape=jax.ShapeDtypeStruct((1, 1), jnp.float32),
        in_specs=[pl.BlockSpec(memory_space=pltpu.VMEM)],
        out_specs=pl.BlockSpec(memory_space=pltpu.SMEM),
    )(z.reshape(NT // 128, 128))
    return out.reshape(1)


def kernel(x, triplets):
    tri = triplets.astype(jnp.int32)
    z = _sc_z(x, tri[:, 0], tri[:, 1], tri[:, 2])
    return _tc_loss(z)
